# baseline (speedup 1.0000x reference)
import functools

import jax
import jax.numpy as jnp
from jax import lax
from jax.experimental import pallas as pl
from jax.experimental.pallas import tpu as pltpu
from jax.experimental.pallas import tpu_sc as plsc

BLOCK_ROWS = 512
N_EXPERTS = 16
TOPK = 2
NC, NS, LANES = 2, 16, 16
NW = NC * NS
CHUNK_ROWS = (8192, 8192)


def _logits_block(x_ref, w_ref, b_ref, logits_ref):
    logits_ref[...] = jnp.dot(x_ref[...], w_ref[...],
                              preferred_element_type=jnp.float32) + b_ref[...]


def _tc_logits_chunk(x, W, b2, base_row, chunk_rows):
    blocks = chunk_rows // BLOCK_ROWS
    base_block = base_row // BLOCK_ROWS
    return pl.pallas_call(
        _logits_block,
        grid=(blocks,),
        in_specs=[
            pl.BlockSpec((BLOCK_ROWS, x.shape[1]),
                         lambda i: (base_block + i, 0)),
            pl.BlockSpec((x.shape[1], N_EXPERTS), lambda i: (0, 0)),
            pl.BlockSpec((1, N_EXPERTS), lambda i: (0, 0)),
        ],
        out_specs=pl.BlockSpec((BLOCK_ROWS, N_EXPERTS), lambda i: (i, 0)),
        out_shape=jax.ShapeDtypeStruct((chunk_rows, N_EXPERTS), jnp.float32),
    )(x, W, b2)


def _sc_route(logits):
    rows = logits.shape[0]
    rpw = rows // NW
    groups = rpw // 8
    mesh = plsc.VectorSubcoreMesh(core_axis_name="c", subcore_axis_name="s",
                                  num_cores=NC, num_subcores=NS)

    @functools.partial(
        pl.kernel,
        out_type=[
            jax.ShapeDtypeStruct((rows, N_EXPERTS), jnp.float32),
            jax.ShapeDtypeStruct((rows * TOPK,), jnp.int32),
        ],
        mesh=mesh,
        scratch_types=[
            pltpu.VMEM((rpw, N_EXPERTS), jnp.float32),
            pltpu.VMEM((rpw * TOPK,), jnp.int32),
        ],
    )
    def route(logits_hbm, gates_hbm, idx_hbm, lg_v, idx_v):
        wid = lax.axis_index("s") * NC + lax.axis_index("c")
        base = wid * rpw
        pltpu.sync_copy(logits_hbm.at[pl.ds(base, rpw)], lg_v)
        lane = lax.iota(jnp.int32, 16)
        neg_inf = jnp.full((16,), -jnp.inf, jnp.float32)
        zero = jnp.zeros((16,), jnp.float32)

        dnums = lax.GatherDimensionNumbers(offset_dims=(),
                                           collapsed_slice_dims=(0,),
                                           start_index_map=(0,))

        def shuffle(v, idx):
            return lax.gather(v, idx[:, None], dnums, (1,),
                              mode=lax.GatherScatterMode.PROMISE_IN_BOUNDS)

        def lane_max(v):
            for k in (1, 2, 4, 8):
                v = jnp.maximum(v, shuffle(v, lane ^ k))
            return v

        def lane_min(v):
            for k in (1, 2, 4, 8):
                v = jnp.minimum(v, shuffle(v, lane ^ k))
            return v

        def argmax_first(v, m):
            return lane_min(jnp.where(v == m, lane, N_EXPERTS))

        def group_body(g, _):
            acc = jnp.zeros((16,), jnp.int32)
            for r in range(8):
                i = g * 8 + r
                v = lg_v[i]
                m1 = lane_max(v)
                i1 = argmax_first(v, m1)
                masked = jnp.where(lane == i1, neg_inf, v)
                m2 = lane_max(masked)
                i2 = argmax_first(masked, m2)
                e = jnp.exp(m2 - m1)
                g2 = e / (1.0 + e)
                g1 = 1.0 - g2
                lg_v[i] = jnp.where(lane == i1, g1,
                                    jnp.where(lane == i2, g2, zero))
                acc = jnp.where(lane == 2 * r, i1, acc)
                acc = jnp.where(lane == 2 * r + 1, i2, acc)
            idx_v[pl.ds(g * 16, 16)] = acc
            return 0

        lax.fori_loop(0, groups, group_body, 0)
        pltpu.sync_copy(lg_v, gates_hbm.at[pl.ds(base, rpw)])
        pltpu.sync_copy(idx_v, idx_hbm.at[pl.ds(base * TOPK, rpw * TOPK)])

    gates, idx_flat = route(logits)
    return gates, idx_flat


@jax.jit
def kernel(x, W, b):
    x = x.astype(jnp.float32)
    Wf = W.astype(jnp.float32)
    rows = x.shape[0]
    b2 = b.reshape(1, N_EXPERTS).astype(jnp.float32)
    gates_parts, idx_parts = [], []
    base_row = 0
    for chunk_rows in CHUNK_ROWS:
        logits_c = _tc_logits_chunk(x, Wf, b2, base_row, chunk_rows)
        gates_c, idx_c = _sc_route(logits_c)
        gates_parts.append(gates_c)
        idx_parts.append(idx_c)
        base_row += chunk_rows
    gates = jnp.concatenate(gates_parts, axis=0)
    idx = jnp.concatenate(idx_parts, axis=0).reshape(rows, TOPK)
    return gates, idx


# speedup vs baseline: 1.2335x; 1.2335x over previous
import functools

import jax
import jax.numpy as jnp
from jax import lax
from jax.experimental import pallas as pl
from jax.experimental.pallas import tpu as pltpu
from jax.experimental.pallas import tpu_sc as plsc

BLOCK_ROWS = 1024
N_EXPERTS = 16
TOPK = 2
NC, NS, LANES = 2, 16, 16
NW = NC * NS
CHUNK_ROWS = (8192, 8192)


def _logits_block(x_ref, w_ref, b_ref, logits_ref):
    logits_ref[...] = jnp.dot(x_ref[...], w_ref[...],
                              preferred_element_type=jnp.float32) + b_ref[...]


def _tc_logits_chunk(x, W, b2, base_row, chunk_rows):
    blocks = chunk_rows // BLOCK_ROWS
    base_block = base_row // BLOCK_ROWS
    return pl.pallas_call(
        _logits_block,
        grid=(blocks,),
        in_specs=[
            pl.BlockSpec((BLOCK_ROWS, x.shape[1]),
                         lambda i: (base_block + i, 0)),
            pl.BlockSpec((x.shape[1], N_EXPERTS), lambda i: (0, 0)),
            pl.BlockSpec((1, N_EXPERTS), lambda i: (0, 0)),
        ],
        out_specs=pl.BlockSpec((BLOCK_ROWS, N_EXPERTS), lambda i: (i, 0)),
        out_shape=jax.ShapeDtypeStruct((chunk_rows, N_EXPERTS), jnp.float32),
    )(x, W, b2)


def _sc_route(logits):
    rows = logits.shape[0]
    rpw = rows // NW
    groups = rpw // 16
    mesh = plsc.VectorSubcoreMesh(core_axis_name="c", subcore_axis_name="s",
                                  num_cores=NC, num_subcores=NS)

    @functools.partial(
        pl.kernel,
        out_type=[
            jax.ShapeDtypeStruct((rows, N_EXPERTS), jnp.float32),
            jax.ShapeDtypeStruct((rows * TOPK,), jnp.int32),
        ],
        mesh=mesh,
        scratch_types=[
            pltpu.VMEM((rpw, N_EXPERTS), jnp.float32),
            pltpu.VMEM((rpw * TOPK,), jnp.int32),
        ],
    )
    def route(logits_hbm, gates_hbm, idx_hbm, lg_v, idx_v):
        wid = lax.axis_index("s") * NC + lax.axis_index("c")
        base = wid * rpw
        pltpu.sync_copy(logits_hbm.at[pl.ds(base, rpw)], lg_v)
        lane = lax.iota(jnp.int32, 16)
        zero = jnp.zeros((16,), jnp.float32)
        zero_i = jnp.zeros((16,), jnp.int32)

        dnums = lax.GatherDimensionNumbers(offset_dims=(),
                                           collapsed_slice_dims=(0,),
                                           start_index_map=(0,))

        def shuffle(v, idx):
            return lax.gather(v, idx[:, None], dnums, (1,),
                              mode=lax.GatherScatterMode.PROMISE_IN_BOUNDS)

        perm = {k: lane ^ k for k in (1, 2, 4, 8)}
        hibit = {k: (lane & k) != 0 for k in (1, 2, 4, 8)}

        def transpose16(v):
            v = list(v)
            for k in (1, 2, 4, 8):
                for r in range(16):
                    if r & k == 0:
                        a, b = v[r], v[r ^ k]
                        v[r] = jnp.where(hibit[k], shuffle(b, perm[k]), a)
                        v[r ^ k] = jnp.where(hibit[k], b, shuffle(a, perm[k]))
            return v

        def group_body(g, _):
            rows = [lg_v[g * 16 + j] for j in range(16)]
            t = transpose16(rows)
            m1, i1 = t[0], zero_i
            m2, i2 = jnp.full((16,), -jnp.inf, jnp.float32), zero_i
            for e in range(1, N_EXPERTS):
                c = t[e]
                gt1 = c > m1
                gt2 = c > m2
                m2 = jnp.where(gt1, m1, jnp.where(gt2, c, m2))
                i2 = jnp.where(gt1, i1, jnp.where(gt2, e, i2))
                m1 = jnp.where(gt1, c, m1)
                i1 = jnp.where(gt1, e, i1)
            ex = jnp.exp(m2 - m1)
            g2 = ex / (1.0 + ex)
            g1 = 1.0 - g2
            cols = [jnp.where(i1 == e, g1, jnp.where(i2 == e, g2, zero))
                    for e in range(N_EXPERTS)]
            gates_rows = transpose16(cols)
            for j in range(16):
                lg_v[g * 16 + j] = gates_rows[j]
            idx_v[pl.ds(g * 16, 16)] = i1
            idx_v[pl.ds(rpw + g * 16, 16)] = i2
            return 0

        lax.fori_loop(0, groups, group_body, 0)
        pltpu.sync_copy(lg_v, gates_hbm.at[pl.ds(base, rpw)])
        pltpu.sync_copy(idx_v, idx_hbm.at[pl.ds(base * TOPK, rpw * TOPK)])

    gates, idx_flat = route(logits)
    idx = idx_flat.reshape(NW, TOPK, rpw).transpose(0, 2, 1).reshape(rows, TOPK)
    return gates, idx


@jax.jit
def kernel(x, W, b):
    x = x.astype(jnp.float32)
    Wf = W.astype(jnp.float32)
    rows = x.shape[0]
    b2 = b.reshape(1, N_EXPERTS).astype(jnp.float32)
    gates_parts, idx_parts = [], []
    base_row = 0
    for chunk_rows in CHUNK_ROWS:
        logits_c = _tc_logits_chunk(x, Wf, b2, base_row, chunk_rows)
        gates_c, idx_c = _sc_route(logits_c)
        gates_parts.append(gates_c)
        idx_parts.append(idx_c)
        base_row += chunk_rows
    gates = jnp.concatenate(gates_parts, axis=0)
    idx = jnp.concatenate(idx_parts, axis=0)
    return gates, idx
